# initial kernel scaffold (unmeasured)
import jax
import jax.numpy as jnp
from jax import lax
from jax.experimental import pallas as pl
from jax.experimental.pallas import tpu as pltpu

N_DEV = 4
B_LOC = 2
SQ = 512
HQ = 32
HQ_LOC = 8
DH = 64
D_MODEL = 768
D_CHUNK = HQ_LOC * DH
BLK = 64
SCALE = 0.125


def kernel(x, Wq, K_ext, V_ext, Wo):
    my = lax.axis_index("i")
    K_loc = lax.dynamic_slice_in_dim(K_ext, my * B_LOC, B_LOC, axis=0)
    V_loc = lax.dynamic_slice_in_dim(V_ext, my * B_LOC, B_LOC, axis=0)
    K_t = jnp.transpose(K_loc, (0, 2, 1, 3))
    V_t = jnp.transpose(V_loc, (0, 2, 1, 3))
    xf = x.reshape(B_LOC * SQ, D_MODEL)

    def body(x_ref, wq_ref, kt_ref, vt_ref, wo_ref, out_ref,
             comm_wq, comm_wo, ctx_ref,
             send_wq, recv_wq, send_wo, recv_wo):
        my_pos = lax.axis_index("i")
        left = lax.rem(my_pos + N_DEV - 1, N_DEV)
        right = lax.rem(my_pos + 1, N_DEV)

        barrier = pltpu.get_barrier_semaphore()
        for nbr in (left, right):
            pl.semaphore_signal(barrier, inc=1, device_id=(nbr,),
                                device_id_type=pl.DeviceIdType.MESH)
        pl.semaphore_wait(barrier, 2)

        qb = lax.broadcasted_iota(jnp.int32, (SQ, SQ), 0) // BLK
        kb = lax.broadcasted_iota(jnp.int32, (SQ, SQ), 1) // BLK
        mask = kb <= qb

        def hop_rdma(src, ref, sends, recvs, h):
            return pltpu.make_async_remote_copy(
                src_ref=src, dst_ref=ref.at[h],
                send_sem=sends.at[h], recv_sem=recvs.at[h],
                device_id=(right,), device_id_type=pl.DeviceIdType.MESH)

        rdmas = []
        for step in range(N_DEV):
            j = lax.rem(my_pos + N_DEV - step, N_DEV)
            if step == 0:
                wq_src, wo_src = wq_ref, wo_ref
            else:
                wq_src, wo_src = comm_wq.at[step - 1], comm_wo.at[step - 1]

            if step < N_DEV - 1:
                r_wq = hop_rdma(wq_src, comm_wq, send_wq, recv_wq, step)
                r_wo = hop_rdma(wo_src, comm_wo, send_wo, recv_wo, step)
                r_wq.start()
                r_wo.start()
                rdmas.append((r_wq, r_wo))

            wq_c = wq_src[...]
            wo_c = wo_src[...]

            qf = lax.dot_general(
                x_ref[...], wq_c, (((1,), (0,)), ((), ())),
                preferred_element_type=jnp.float32)

            for b in range(B_LOC):
                kc = kt_ref[b, pl.ds(j * HQ_LOC, HQ_LOC)]
                vc = vt_ref[b, pl.ds(j * HQ_LOC, HQ_LOC)]
                for h in range(HQ_LOC):
                    q_bh = qf[b * SQ:(b + 1) * SQ, h * DH:(h + 1) * DH]
                    s = lax.dot_general(
                        q_bh, kc[h], (((1,), (1,)), ((), ())),
                        preferred_element_type=jnp.float32)
                    s = jnp.where(mask, s * SCALE, -1e9)
                    m = jnp.max(s, axis=1, keepdims=True)
                    w = jnp.exp(s - m)
                    w = w / jnp.sum(w, axis=1, keepdims=True)
                    c_bh = lax.dot_general(
                        w, vc[h], (((1,), (0,)), ((), ())),
                        preferred_element_type=jnp.float32)
                    ctx_ref[b * SQ:(b + 1) * SQ, h * DH:(h + 1) * DH] = c_bh

            contrib = lax.dot_general(
                ctx_ref[...], wo_c, (((1,), (0,)), ((), ())),
                preferred_element_type=jnp.float32)
            if step == 0:
                out_ref[...] = contrib
            else:
                out_ref[...] = out_ref[...] + contrib

            if step < N_DEV - 1:
                r_wq, r_wo = rdmas[-1]
                r_wq.wait_recv()
                r_wo.wait_recv()

        for r_wq, r_wo in rdmas:
            r_wq.wait_send()
            r_wo.wait_send()

    out = pl.pallas_call(
        body,
        out_shape=jax.ShapeDtypeStruct((B_LOC * SQ, D_MODEL), jnp.float32),
        in_specs=[pl.BlockSpec(memory_space=pltpu.VMEM)] * 5,
        out_specs=pl.BlockSpec(memory_space=pltpu.VMEM),
        scratch_shapes=[
            pltpu.VMEM((N_DEV - 1, D_MODEL, D_CHUNK), jnp.float32),
            pltpu.VMEM((N_DEV - 1, D_CHUNK, D_MODEL), jnp.float32),
            pltpu.VMEM((B_LOC * SQ, D_CHUNK), jnp.float32),
            pltpu.SemaphoreType.DMA((N_DEV - 1,)),
            pltpu.SemaphoreType.DMA((N_DEV - 1,)),
            pltpu.SemaphoreType.DMA((N_DEV - 1,)),
            pltpu.SemaphoreType.DMA((N_DEV - 1,)),
        ],
        compiler_params=pltpu.CompilerParams(collective_id=0),
    )(xf, Wq, K_t, V_t, Wo)
    return out.reshape(B_LOC, SQ, D_MODEL)


# baseline (device time: 163073 ns/iter reference)
import jax
import jax.numpy as jnp
from jax import lax
from jax.experimental import pallas as pl
from jax.experimental.pallas import tpu as pltpu

N_DEV = 4
B_LOC = 2
SQ = 512
HQ = 32
HQ_LOC = 8
DH = 64
D_MODEL = 768
D_CHUNK = HQ_LOC * DH
BLK = 64
SCALE = 0.125


def kernel(x, Wq, K_ext, V_ext, Wo):
    my = lax.axis_index("i")
    K_loc = lax.dynamic_slice_in_dim(K_ext, my * B_LOC, B_LOC, axis=0)
    V_loc = lax.dynamic_slice_in_dim(V_ext, my * B_LOC, B_LOC, axis=0)
    K_t = jnp.transpose(K_loc, (0, 2, 1, 3))
    V_t = jnp.transpose(V_loc, (0, 2, 1, 3))
    xf = x.reshape(B_LOC * SQ, D_MODEL)

    def body(x_ref, wq_ref, kt_ref, vt_ref, wo_ref, out_ref,
             comm_wq, comm_wo, ctx_ref,
             send_wq, recv_wq, send_wo, recv_wo):
        my_pos = lax.axis_index("i")
        left = lax.rem(my_pos + N_DEV - 1, N_DEV)
        right = lax.rem(my_pos + 1, N_DEV)

        barrier = pltpu.get_barrier_semaphore()
        for nbr in (left, right):
            pl.semaphore_signal(barrier, inc=1, device_id=(nbr,),
                                device_id_type=pl.DeviceIdType.MESH)
        pl.semaphore_wait(barrier, 2)

        qb = lax.broadcasted_iota(jnp.int32, (SQ, SQ), 0) // BLK
        kb = lax.broadcasted_iota(jnp.int32, (SQ, SQ), 1) // BLK
        mask = kb <= qb

        def hop_rdma(src, ref, sends, recvs, h):
            return pltpu.make_async_remote_copy(
                src_ref=src, dst_ref=ref.at[h],
                send_sem=sends.at[h], recv_sem=recvs.at[h],
                device_id=(right,), device_id_type=pl.DeviceIdType.MESH)

        rdmas = []
        for step in range(N_DEV):
            j = lax.rem(my_pos + N_DEV - step, N_DEV)
            if step == 0:
                wq_src, wo_src = wq_ref, wo_ref
            else:
                wq_src, wo_src = comm_wq.at[step - 1], comm_wo.at[step - 1]

            if step < N_DEV - 1:
                r_wq = hop_rdma(wq_src, comm_wq, send_wq, recv_wq, step)
                r_wo = hop_rdma(wo_src, comm_wo, send_wo, recv_wo, step)
                r_wq.start()
                r_wo.start()
                rdmas.append((r_wq, r_wo))

            wq_c = wq_src[...]
            wo_c = wo_src[...]

            qf = lax.dot_general(
                x_ref[...], wq_c, (((1,), (0,)), ((), ())),
                preferred_element_type=jnp.float32)

            for b in range(B_LOC):
                kc = kt_ref[b, pl.ds(j * HQ_LOC, HQ_LOC)]
                vc = vt_ref[b, pl.ds(j * HQ_LOC, HQ_LOC)]
                for h in range(HQ_LOC):
                    q_bh = qf[b * SQ:(b + 1) * SQ, h * DH:(h + 1) * DH]
                    s = lax.dot_general(
                        q_bh, kc[h], (((1,), (1,)), ((), ())),
                        preferred_element_type=jnp.float32)
                    s = jnp.where(mask, s * SCALE, -1e9)
                    m = jnp.max(s, axis=1, keepdims=True)
                    w = jnp.exp(s - m)
                    w = w / jnp.sum(w, axis=1, keepdims=True)
                    c_bh = lax.dot_general(
                        w, vc[h], (((1,), (0,)), ((), ())),
                        preferred_element_type=jnp.float32)
                    ctx_ref[b * SQ:(b + 1) * SQ, h * DH:(h + 1) * DH] = c_bh

            contrib = lax.dot_general(
                ctx_ref[...], wo_c, (((1,), (0,)), ((), ())),
                preferred_element_type=jnp.float32)
            if step == 0:
                out_ref[...] = contrib
            else:
                out_ref[...] = out_ref[...] + contrib

            if step < N_DEV - 1:
                r_wq, r_wo = rdmas[-1]
                r_wq.wait_recv()
                r_wo.wait_recv()

        for r_wq, r_wo in rdmas:
            r_wq.wait_send()
            r_wo.wait_send()

    out = pl.pallas_call(
        body,
        out_shape=jax.ShapeDtypeStruct((B_LOC * SQ, D_MODEL), jnp.float32),
        in_specs=[pl.BlockSpec(memory_space=pltpu.VMEM)] * 5,
        out_specs=pl.BlockSpec(memory_space=pltpu.VMEM),
        scratch_shapes=[
            pltpu.VMEM((N_DEV - 1, D_MODEL, D_CHUNK), jnp.float32),
            pltpu.VMEM((N_DEV - 1, D_CHUNK, D_MODEL), jnp.float32),
            pltpu.VMEM((B_LOC * SQ, D_CHUNK), jnp.float32),
            pltpu.SemaphoreType.DMA((N_DEV - 1,)),
            pltpu.SemaphoreType.DMA((N_DEV - 1,)),
            pltpu.SemaphoreType.DMA((N_DEV - 1,)),
            pltpu.SemaphoreType.DMA((N_DEV - 1,)),
        ],
        compiler_params=pltpu.CompilerParams(
            collective_id=0, vmem_limit_bytes=100 * 1024 * 1024),
    )(xf, Wq, K_t, V_t, Wo)
    return out.reshape(B_LOC, SQ, D_MODEL)


# device time: 77270 ns/iter; 2.1104x vs baseline; 2.1104x over previous
import jax
import jax.numpy as jnp
from jax import lax
from jax.experimental import pallas as pl
from jax.experimental.pallas import tpu as pltpu

N_DEV = 4
B_LOC = 2
SQ = 512
HQ_LOC = 8
DH = 64
D_MODEL = 768
D_CHUNK = HQ_LOC * DH
BLK = 64
SCALE = 0.125


def kernel(x, Wq, K_ext, V_ext, Wo):
    my = lax.axis_index("i")
    K_loc = lax.dynamic_slice_in_dim(K_ext, my * B_LOC, B_LOC, axis=0)
    V_loc = lax.dynamic_slice_in_dim(V_ext, my * B_LOC, B_LOC, axis=0)
    K_t = jnp.transpose(K_loc, (0, 2, 1, 3)).astype(jnp.bfloat16)
    V_t = jnp.transpose(V_loc, (0, 2, 1, 3)).astype(jnp.bfloat16)
    xf = x.reshape(B_LOC * SQ, D_MODEL).astype(jnp.bfloat16)
    Wq_b = Wq.astype(jnp.bfloat16)
    Wo_b = Wo.astype(jnp.bfloat16)

    def body(x_ref, wq_ref, kt_ref, vt_ref, wo_ref, out_ref,
             comm_wq, comm_wo, ctx_ref,
             send_wq, recv_wq, send_wo, recv_wo):
        my_pos = lax.axis_index("i")
        left = lax.rem(my_pos + N_DEV - 1, N_DEV)
        right = lax.rem(my_pos + 1, N_DEV)

        barrier = pltpu.get_barrier_semaphore()
        for nbr in (left, right):
            pl.semaphore_signal(barrier, inc=1, device_id=(nbr,),
                                device_id_type=pl.DeviceIdType.MESH)
        pl.semaphore_wait(barrier, 2)

        qb = lax.broadcasted_iota(jnp.int32, (SQ, SQ), 0) // BLK
        kb = lax.broadcasted_iota(jnp.int32, (SQ, SQ), 1) // BLK
        bias = jnp.where(kb <= qb, 0.0, -1e9).astype(jnp.float32)

        def hop_rdma(src, ref, sems_s, sems_r, h, tgt):
            return pltpu.make_async_remote_copy(
                src_ref=src, dst_ref=ref.at[h],
                send_sem=sems_s.at[h], recv_sem=sems_r.at[h],
                device_id=(tgt,), device_id_type=pl.DeviceIdType.MESH)

        def attention(step, j, wq_c):
            qf = lax.dot_general(
                x_ref[...], wq_c, (((1,), (0,)), ((), ())),
                preferred_element_type=jnp.float32).astype(jnp.bfloat16)
            for b in range(B_LOC):
                kc = kt_ref[b, pl.ds(j * HQ_LOC, HQ_LOC)]
                vc = vt_ref[b, pl.ds(j * HQ_LOC, HQ_LOC)]
                for h in range(HQ_LOC):
                    q_bh = qf[b * SQ:(b + 1) * SQ, h * DH:(h + 1) * DH]
                    s = lax.dot_general(
                        q_bh, kc[h], (((1,), (1,)), ((), ())),
                        preferred_element_type=jnp.float32)
                    w = jnp.exp(s * SCALE + bias)
                    wsum = jnp.sum(w, axis=1, keepdims=True)
                    c_bh = lax.dot_general(
                        w.astype(jnp.bfloat16), vc[h], (((1,), (0,)), ((), ())),
                        preferred_element_type=jnp.float32)
                    c_bh = (c_bh / wsum).astype(jnp.bfloat16)
                    ctx_ref[b * SQ:(b + 1) * SQ,
                            step * D_CHUNK + h * DH:
                            step * D_CHUNK + (h + 1) * DH] = c_bh

        def out_proj(slot, wo_c, first=False):
            contrib = lax.dot_general(
                ctx_ref[:, slot * D_CHUNK:(slot + 1) * D_CHUNK], wo_c,
                (((1,), (0,)), ((), ())),
                preferred_element_type=jnp.float32)
            if first:
                out_ref[...] = contrib
            else:
                out_ref[...] = out_ref[...] + contrib

        rdmas = []
        r_wq = hop_rdma(wq_ref, comm_wq, send_wq, recv_wq, 0, right)
        r_wo = hop_rdma(wo_ref, comm_wo, send_wo, recv_wo, 0, left)
        r_wq.start()
        r_wo.start()
        rdmas.append((r_wq, r_wo))

        attention(0, my_pos, wq_ref[...])
        out_proj(0, wo_ref[...], first=True)

        for s in range(1, N_DEV):
            r_wq, r_wo = rdmas[-1]
            r_wq.wait_recv()
            r_wo.wait_recv()
            if s < N_DEV - 1:
                n_wq = hop_rdma(comm_wq.at[s - 1], comm_wq, send_wq, recv_wq,
                                s, right)
                n_wo = hop_rdma(comm_wo.at[s - 1], comm_wo, send_wo, recv_wo,
                                s, left)
                n_wq.start()
                n_wo.start()
                rdmas.append((n_wq, n_wo))

            j = lax.rem(my_pos + N_DEV - s, N_DEV)
            attention(s, j, comm_wq[s - 1])
            if s == 2:
                out_proj(2, comm_wo[1])
            elif s == 3:
                out_proj(3, comm_wo[0])
                out_proj(1, comm_wo[2])

        for r_wq, r_wo in rdmas:
            r_wq.wait_send()
            r_wo.wait_send()

    out = pl.pallas_call(
        body,
        out_shape=jax.ShapeDtypeStruct((B_LOC * SQ, D_MODEL), jnp.float32),
        in_specs=[pl.BlockSpec(memory_space=pltpu.VMEM)] * 5,
        out_specs=pl.BlockSpec(memory_space=pltpu.VMEM),
        scratch_shapes=[
            pltpu.VMEM((N_DEV - 1, D_MODEL, D_CHUNK), jnp.bfloat16),
            pltpu.VMEM((N_DEV - 1, D_CHUNK, D_MODEL), jnp.bfloat16),
            pltpu.VMEM((B_LOC * SQ, N_DEV * D_CHUNK), jnp.bfloat16),
            pltpu.SemaphoreType.DMA((N_DEV - 1,)),
            pltpu.SemaphoreType.DMA((N_DEV - 1,)),
            pltpu.SemaphoreType.DMA((N_DEV - 1,)),
            pltpu.SemaphoreType.DMA((N_DEV - 1,)),
        ],
        compiler_params=pltpu.CompilerParams(
            collective_id=0, vmem_limit_bytes=100 * 1024 * 1024),
    )(xf, Wq_b, K_t, V_t, Wo_b)
    return out.reshape(B_LOC, SQ, D_MODEL)
